# Initial kernel scaffold; baseline (speedup 1.0000x reference)
import functools

import jax
import jax.numpy as jnp
from jax import lax
from jax.experimental import pallas as pl
from jax.experimental.pallas import tpu as pltpu
from jax.experimental.pallas import tpu_sc as plsc

B = 4096
L = 200
EMB = 32
N = B * L

_info = plsc.get_sparse_core_info()
NC = _info.num_cores
NS = _info.num_subcores
NW = NC * NS
PER_W = N // NW
CH = 512
NCHUNK = PER_W // CH

_mesh = plsc.VectorSubcoreMesh(core_axis_name="c", subcore_axis_name="s")


@functools.partial(
    pl.kernel,
    mesh=_mesh,
    out_type=jax.ShapeDtypeStruct((N, 2, EMB), jnp.float32),
    scratch_types=[
        pltpu.VMEM((CH,), jnp.int32),
        pltpu.VMEM((CH,), jnp.int32),
        pltpu.VMEM((CH, EMB), jnp.float32),
        pltpu.VMEM((CH, EMB), jnp.float32),
        pltpu.SemaphoreType.DMA,
        pltpu.SemaphoreType.DMA,
    ],
)
def _sc_gather(idx0_hbm, idx1_hbm, t0_hbm, t1_hbm, out_hbm,
               idx0_v, idx1_v, r0_v, r1_v, s0, s1):
    wid = lax.axis_index("s") * NC + lax.axis_index("c")

    def body(c, carry):
        base = wid * PER_W + c * CH
        pltpu.sync_copy(idx0_hbm.at[pl.ds(base, CH)], idx0_v)
        pltpu.sync_copy(idx1_hbm.at[pl.ds(base, CH)], idx1_v)
        cp0 = pltpu.async_copy(t0_hbm.at[idx0_v], r0_v, s0)
        cp1 = pltpu.async_copy(t1_hbm.at[idx1_v], r1_v, s1)
        cp0.wait()
        cp1.wait()
        pltpu.sync_copy(r0_v, out_hbm.at[pl.ds(base, CH), 0])
        pltpu.sync_copy(r1_v, out_hbm.at[pl.ds(base, CH), 1])
        return carry

    lax.fori_loop(0, NCHUNK, body, 0)


def _mask_body(h_ref, m_ref):
    m_ref[...] = h_ref[...] != 0


_mask_call = pl.pallas_call(
    _mask_body,
    out_shape=jax.ShapeDtypeStruct((B, L), jnp.bool_),
)


def kernel(histories, item_emb_0, item_emb_1):
    hist0 = histories[:, 0, :]
    idx0 = hist0.reshape(N)
    idx1 = histories[:, 1, :].reshape(N)
    embs = _sc_gather(idx0, idx1, item_emb_0, item_emb_1)
    mask = _mask_call(hist0)
    return embs.reshape(B, L, 2 * EMB), mask


# speedup vs baseline: 1.7759x; 1.7759x over previous
import functools

import jax
import jax.numpy as jnp
from jax import lax
from jax.experimental import pallas as pl
from jax.experimental.pallas import tpu as pltpu
from jax.experimental.pallas import tpu_sc as plsc

B = 4096
L = 200
EMB = 32
N = B * L

_info = plsc.get_sparse_core_info()
NC = _info.num_cores
NS = _info.num_subcores
NW = NC * NS
PER_W = N // NW
CH = 512
NCHUNK = PER_W // CH

_mesh = plsc.VectorSubcoreMesh(core_axis_name="c", subcore_axis_name="s")


@functools.partial(
    pl.kernel,
    mesh=_mesh,
    compiler_params=pltpu.CompilerParams(use_tc_tiling_on_sc=False),
    out_type=jax.ShapeDtypeStruct((N, 2, EMB), jnp.float32),
    scratch_types=[
        pltpu.VMEM((CH,), jnp.int32),
        pltpu.VMEM((CH,), jnp.int32),
        pltpu.VMEM((CH, EMB), jnp.float32),
        pltpu.VMEM((CH, EMB), jnp.float32),
        pltpu.SemaphoreType.DMA,
        pltpu.SemaphoreType.DMA,
    ],
)
def _sc_gather(idx0_hbm, idx1_hbm, t0_hbm, t1_hbm, out_hbm,
               idx0_v, idx1_v, r0_v, r1_v, s0, s1):
    wid = lax.axis_index("s") * NC + lax.axis_index("c")

    def body(c, carry):
        base = wid * PER_W + c * CH
        pltpu.sync_copy(idx0_hbm.at[pl.ds(base, CH)], idx0_v)
        pltpu.sync_copy(idx1_hbm.at[pl.ds(base, CH)], idx1_v)
        cp0 = pltpu.async_copy(t0_hbm.at[idx0_v], r0_v, s0)
        cp1 = pltpu.async_copy(t1_hbm.at[idx1_v], r1_v, s1)
        cp0.wait()
        cp1.wait()
        pltpu.sync_copy(r0_v, out_hbm.at[pl.ds(base, CH), 0])
        pltpu.sync_copy(r1_v, out_hbm.at[pl.ds(base, CH), 1])
        return carry

    lax.fori_loop(0, NCHUNK, body, 0)


def _mask_body(h_ref, m_ref):
    m_ref[...] = h_ref[...] != 0


_mask_call = pl.pallas_call(
    _mask_body,
    out_shape=jax.ShapeDtypeStruct((B, L), jnp.bool_),
)


def kernel(histories, item_emb_0, item_emb_1):
    hist0 = histories[:, 0, :]
    idx0 = hist0.reshape(N)
    idx1 = histories[:, 1, :].reshape(N)
    embs = _sc_gather(idx0, idx1, item_emb_0, item_emb_1)
    mask = _mask_call(hist0)
    return embs.reshape(B, L, 2 * EMB), mask


# speedup vs baseline: 1.8870x; 1.0625x over previous
import functools

import jax
import jax.numpy as jnp
from jax import lax
from jax.experimental import pallas as pl
from jax.experimental.pallas import tpu as pltpu
from jax.experimental.pallas import tpu_sc as plsc

B = 4096
L = 200
EMB = 32
N = B * L

_info = plsc.get_sparse_core_info()
NC = _info.num_cores
NS = _info.num_subcores
NW = NC * NS
PER_W = N // NW
CH = 256
NCHUNK = PER_W // CH
NSLOT = 4

_mesh = plsc.VectorSubcoreMesh(core_axis_name="c", subcore_axis_name="s")


@functools.partial(
    pl.kernel,
    mesh=_mesh,
    compiler_params=pltpu.CompilerParams(use_tc_tiling_on_sc=False),
    out_type=jax.ShapeDtypeStruct((N, 2, EMB), jnp.float32),
    scratch_types=[
        pltpu.VMEM((PER_W,), jnp.int32),
        pltpu.VMEM((PER_W,), jnp.int32),
        [pltpu.VMEM((CH, EMB), jnp.float32) for _ in range(NSLOT)],
        [pltpu.VMEM((CH, EMB), jnp.float32) for _ in range(NSLOT)],
        [pltpu.SemaphoreType.DMA for _ in range(NSLOT)],
        [pltpu.SemaphoreType.DMA for _ in range(NSLOT)],
    ],
)
def _sc_gather(idx0_hbm, idx1_hbm, t0_hbm, t1_hbm, out_hbm,
               idx0_v, idx1_v, r0, r1, gsem, wsem):
    wid = lax.axis_index("s") * NC + lax.axis_index("c")
    base = wid * PER_W

    pltpu.sync_copy(idx0_hbm.at[pl.ds(base, PER_W)], idx0_v)
    pltpu.sync_copy(idx1_hbm.at[pl.ds(base, PER_W)], idx1_v)

    def fire_gathers(c, p):
        pltpu.async_copy(t0_hbm.at[idx0_v.at[pl.ds(c * CH, CH)]], r0[p], gsem[p])
        pltpu.async_copy(t1_hbm.at[idx1_v.at[pl.ds(c * CH, CH)]], r1[p], gsem[p])

    def wait_gathers(p):
        pltpu.make_async_copy(t0_hbm.at[pl.ds(0, CH)], r0[p], gsem[p]).wait()
        pltpu.make_async_copy(t1_hbm.at[pl.ds(0, CH)], r1[p], gsem[p]).wait()

    def fire_write(c, p):
        pltpu.async_copy(r0[p], out_hbm.at[pl.ds(base + c * CH, CH), 0], wsem[p])
        pltpu.async_copy(r1[p], out_hbm.at[pl.ds(base + c * CH, CH), 1], wsem[p])

    def wait_write(p):
        pltpu.make_async_copy(t0_hbm.at[pl.ds(0, CH)], r0[p], wsem[p]).wait()
        pltpu.make_async_copy(t1_hbm.at[pl.ds(0, CH)], r1[p], wsem[p]).wait()

    for p in range(NSLOT):
        fire_gathers(p, p)

    def steady(r, carry):
        for p in range(NSLOT):
            c = r * NSLOT + p
            wait_gathers(p)
            fire_write(c - NSLOT, p)
            wait_write(p)
            fire_gathers(c, p)
        return carry

    lax.fori_loop(1, NCHUNK // NSLOT, steady, 0)

    for i in range(NSLOT):
        c = NCHUNK - NSLOT + i
        p = c % NSLOT
        wait_gathers(p)
        fire_write(c, p)
    for i in range(NSLOT):
        p = (NCHUNK - NSLOT + i) % NSLOT
        wait_write(p)


def _mask_body(h_ref, m_ref):
    m_ref[...] = h_ref[...] != 0


_mask_call = pl.pallas_call(
    _mask_body,
    out_shape=jax.ShapeDtypeStruct((B, L), jnp.bool_),
)


def kernel(histories, item_emb_0, item_emb_1):
    hist0 = histories[:, 0, :]
    idx0 = hist0.reshape(N)
    idx1 = histories[:, 1, :].reshape(N)
    embs = _sc_gather(idx0, idx1, item_emb_0, item_emb_1)
    mask = _mask_call(hist0)
    return embs.reshape(B, L, 2 * EMB), mask
